# Initial kernel scaffold; baseline (speedup 1.0000x reference)
import functools

import jax
import jax.numpy as jnp
from jax import lax
from jax.experimental import pallas as pl
from jax.experimental.pallas import tpu as pltpu
from jax.experimental.pallas import tpu_sc as plsc

N = 10000
E = 320000
D = 128

NC = 2
NS = 16
L = 16

NPAD = 10240
RPT = NPAD // NS

EPT_DEG = E // NS

_mesh = plsc.VectorSubcoreMesh(core_axis_name="c", subcore_axis_name="s")


@functools.partial(
    pl.kernel,
    out_type=jax.ShapeDtypeStruct((NC, NPAD), jnp.float32),
    mesh=_mesh,
    scratch_types=[
        pltpu.VMEM((EPT_DEG,), jnp.int32),
        pltpu.VMEM((NPAD,), jnp.float32),
        pltpu.VMEM((RPT,), jnp.float32),
        pltpu.VMEM((RPT,), jnp.float32),
        pltpu.VMEM_SHARED((NS, NPAD), jnp.float32),
    ],
)
def _deg_kernel(edges_hbm, deg_hbm, idx_v, hist_v, acc_v, tmp_v, shared):
    c = lax.axis_index("c")
    s = lax.axis_index("s")

    zeros16 = jnp.zeros((L,), jnp.float32)
    ones16 = jnp.ones((L,), jnp.float32)

    def zero_hist(i, carry):
        hist_v[pl.ds(i * L, L)] = zeros16
        return carry

    lax.fori_loop(0, NPAD // L, zero_hist, 0)

    pltpu.sync_copy(edges_hbm.at[c, pl.ds(s * EPT_DEG, EPT_DEG)], idx_v)

    def accum(i, carry):
        idx = idx_v[pl.ds(i * L, L)]
        plsc.addupdate_scatter(hist_v, [idx], ones16)
        return carry

    lax.fori_loop(0, EPT_DEG // L, accum, 0)

    pltpu.sync_copy(hist_v, shared.at[s])
    plsc.subcore_barrier()

    def zero_acc(i, carry):
        acc_v[pl.ds(i * L, L)] = zeros16
        return carry

    lax.fori_loop(0, RPT // L, zero_acc, 0)

    def combine(k, carry):
        pltpu.sync_copy(shared.at[k, pl.ds(s * RPT, RPT)], tmp_v)

        def add_vec(i, carry2):
            acc_v[pl.ds(i * L, L)] = acc_v[pl.ds(i * L, L)] + tmp_v[pl.ds(i * L, L)]
            return carry2

        lax.fori_loop(0, RPT // L, add_vec, 0)
        return carry

    lax.fori_loop(0, NS, combine, 0)

    pltpu.sync_copy(acc_v, deg_hbm.at[c, pl.ds(s * RPT, RPT)])


RMM = 1000


def _mm_body(f_ref, w_ref, deg_ref, xw_ref):
    scale = lax.rsqrt(jnp.maximum(deg_ref[...], 1.0))
    xw_ref[...] = jnp.dot(f_ref[...], w_ref[...],
                          preferred_element_type=jnp.float32) * scale


def _mm(features, W, deg_out2d):
    return pl.pallas_call(
        _mm_body,
        grid=(N // RMM,),
        in_specs=[
            pl.BlockSpec((RMM, D), lambda i: (i, 0)),
            pl.BlockSpec((D, D), lambda i: (0, 0)),
            pl.BlockSpec((RMM, 1), lambda i: (i, 0)),
        ],
        out_specs=pl.BlockSpec((RMM, D), lambda i: (i, 0)),
        out_shape=jax.ShapeDtypeStruct((N, D), jnp.float32),
    )(features, W, deg_out2d)


B = 80
EPS = E // (NC * NS)
NB = EPS // B


@functools.partial(
    pl.kernel,
    out_type=[
        jax.ShapeDtypeStruct((NPAD, D), jnp.float32),
        jax.ShapeDtypeStruct((NPAD, D), jnp.float32),
    ],
    mesh=_mesh,
    scratch_types=[
        pltpu.VMEM((NB, B), jnp.int32),
        pltpu.VMEM((NB, B), jnp.int32),
        pltpu.VMEM((B, D), jnp.float32),
        pltpu.VMEM((B, D), jnp.float32),
        pltpu.VMEM_SHARED((NPAD, D), jnp.float32),
        pltpu.SemaphoreType.DMA,
    ],
)
def _gs_kernel(xw_hbm, src_hbm, dst_hbm, p0_hbm, p1_hbm,
               src_v, dst_v, rows_v, zb_v, shared, sem):
    c = lax.axis_index("c")
    s = lax.axis_index("s")

    zeros16 = jnp.zeros((L,), jnp.float32)

    def zero_zb(t, carry):
        i = t // (D // L)
        j = t % (D // L)
        zb_v[i, pl.ds(j * L, L)] = zeros16
        return carry

    lax.fori_loop(0, B * D // L, zero_zb, 0)

    for k in range(RPT // B):
        pltpu.sync_copy(zb_v, shared.at[pl.ds(s * RPT + k * B, B)])
    plsc.subcore_barrier()

    rowbase = c * (E // NC // B) + s * NB
    pltpu.sync_copy(src_hbm.at[pl.ds(rowbase, NB)], src_v)
    pltpu.sync_copy(dst_hbm.at[pl.ds(rowbase, NB)], dst_v)

    def body(j, carry):
        pltpu.async_copy(xw_hbm.at[src_v.at[j]], rows_v, sem).wait()
        pltpu.sync_copy(rows_v, shared.at[dst_v.at[j]], add=True)
        return carry

    lax.fori_loop(0, NB, body, 0)
    plsc.subcore_barrier()

    @pl.when(c == 0)
    def _():
        pltpu.sync_copy(shared.at[pl.ds(s * RPT, RPT)],
                        p0_hbm.at[pl.ds(s * RPT, RPT)])

    @pl.when(c == 1)
    def _():
        pltpu.sync_copy(shared.at[pl.ds(s * RPT, RPT)],
                        p1_hbm.at[pl.ds(s * RPT, RPT)])


def _ep_body(p0_ref, p1_ref, deg_ref, b_ref, out_ref):
    scale = lax.rsqrt(jnp.maximum(deg_ref[...], 1.0))
    agg = (p0_ref[...] + p1_ref[...]) * scale
    out_ref[...] = jnp.maximum(agg + b_ref[...], 0.0)


def _epilogue(p0, p1, deg_in2d, b2d):
    return pl.pallas_call(
        _ep_body,
        grid=(N // RMM,),
        in_specs=[
            pl.BlockSpec((RMM, D), lambda i: (i, 0)),
            pl.BlockSpec((RMM, D), lambda i: (i, 0)),
            pl.BlockSpec((RMM, 1), lambda i: (i, 0)),
            pl.BlockSpec((1, D), lambda i: (0, 0)),
        ],
        out_specs=pl.BlockSpec((RMM, D), lambda i: (i, 0)),
        out_shape=jax.ShapeDtypeStruct((N, D), jnp.float32),
    )(p0, p1, deg_in2d, b2d)


def kernel(features, edge_index, W, b):
    deg = _deg_kernel(edge_index)
    deg_out2d = deg[0, :N, None]
    deg_in2d = deg[1, :N, None]
    xw = _mm(features, W, deg_out2d)
    src2 = edge_index[0].reshape(E // B, B)
    dst2 = edge_index[1].reshape(E // B, B)
    p0, p1 = _gs_kernel(xw, src2, dst2)
    return _epilogue(p0, p1, deg_in2d, b[None, :])


# speedup vs baseline: 13.4292x; 13.4292x over previous
import functools

import jax
import jax.numpy as jnp
from jax import lax
from jax.experimental import pallas as pl
from jax.experimental.pallas import tpu as pltpu
from jax.experimental.pallas import tpu_sc as plsc

N = 10000
E = 320000
D = 128

NC = 2
NS = 16
L = 16

_mesh = plsc.VectorSubcoreMesh(core_axis_name="c", subcore_axis_name="s")
_sc_params = pltpu.CompilerParams(needs_layout_passes=False)

HSZ = 16384
EPT_DEG = E // NS
HPT = HSZ // NS


@functools.partial(
    pl.kernel,
    out_type=[
        jax.ShapeDtypeStruct((HSZ,), jnp.float32),
        jax.ShapeDtypeStruct((HSZ,), jnp.float32),
    ],
    mesh=_mesh,
    scratch_types=[
        pltpu.VMEM((EPT_DEG,), jnp.int32),
        pltpu.VMEM((HSZ,), jnp.float32),
        pltpu.VMEM((HPT,), jnp.float32),
        pltpu.VMEM((HPT,), jnp.float32),
        pltpu.VMEM_SHARED((NS * HSZ,), jnp.float32),
    ],
    compiler_params=_sc_params,
)
def _deg_kernel(src_hbm, dst_hbm, dsrc_hbm, ddst_hbm,
                idx_v, hist_v, acc_v, tmp_v, shared):
    c = lax.axis_index("c")
    s = lax.axis_index("s")

    zeros16 = jnp.zeros((L,), jnp.float32)
    ones16 = jnp.ones((L,), jnp.float32)

    def zero_hist(i, carry):
        hist_v[pl.ds(i * L, L)] = zeros16
        return carry

    lax.fori_loop(0, HSZ // L, zero_hist, 0)

    @pl.when(c == 0)
    def _():
        pltpu.sync_copy(src_hbm.at[pl.ds(s * EPT_DEG, EPT_DEG)], idx_v)

    @pl.when(c == 1)
    def _():
        pltpu.sync_copy(dst_hbm.at[pl.ds(s * EPT_DEG, EPT_DEG)], idx_v)

    def accum(i, carry):
        idx = idx_v[pl.ds(i * L, L)]
        plsc.addupdate_scatter(hist_v, [idx], ones16)
        return carry

    lax.fori_loop(0, EPT_DEG // L, accum, 0)

    pltpu.sync_copy(hist_v, shared.at[pl.ds(s * HSZ, HSZ)])
    plsc.subcore_barrier()

    def zero_acc(i, carry):
        acc_v[pl.ds(i * L, L)] = zeros16
        return carry

    lax.fori_loop(0, HPT // L, zero_acc, 0)

    def combine(k, carry):
        pltpu.sync_copy(shared.at[pl.ds(k * HSZ + s * HPT, HPT)], tmp_v)

        def add_vec(i, carry2):
            j = i * L
            acc_v[pl.ds(j, L)] = acc_v[pl.ds(j, L)] + tmp_v[pl.ds(j, L)]
            return carry2

        lax.fori_loop(0, HPT // L, add_vec, 0)
        return carry

    lax.fori_loop(0, NS, combine, 0)

    @pl.when(c == 0)
    def _():
        pltpu.sync_copy(acc_v, dsrc_hbm.at[pl.ds(s * HPT, HPT)])

    @pl.when(c == 1)
    def _():
        pltpu.sync_copy(acc_v, ddst_hbm.at[pl.ds(s * HPT, HPT)])


RMM = 1000


def _mm_body(f_ref, w_ref, deg_ref, xw_ref):
    scale = lax.rsqrt(jnp.maximum(deg_ref[...], 1.0))
    xw_ref[...] = jnp.dot(f_ref[...], w_ref[...],
                          preferred_element_type=jnp.float32) * scale


def _mm(features, W, deg_out2d):
    return pl.pallas_call(
        _mm_body,
        grid=(N // RMM,),
        in_specs=[
            pl.BlockSpec((RMM, D), lambda i: (i, 0)),
            pl.BlockSpec((D, D), lambda i: (0, 0)),
            pl.BlockSpec((RMM, 1), lambda i: (i, 0)),
        ],
        out_specs=pl.BlockSpec((RMM, D), lambda i: (i, 0)),
        out_shape=jax.ShapeDtypeStruct((N, D), jnp.float32),
    )(features, W, deg_out2d)


GB = 80
GR = 8
GE = GB * GR
NG = E // GE
GPC = NG // NC
HALF = 5000
AGG = 6144
TRASH = 5632
ZR = 96
OPT = 5120 // NS


@functools.partial(
    pl.kernel,
    out_type=jax.ShapeDtypeStruct((2, NC, 5120, D), jnp.float32),
    mesh=_mesh,
    scratch_types=[
        pltpu.VMEM((GR, GB), jnp.int32),
        pltpu.VMEM((GR, GB), jnp.int32),
        pltpu.VMEM((GB, D), jnp.float32),
        pltpu.VMEM((ZR, D), jnp.float32),
        pltpu.VMEM_SHARED((AGG, D), jnp.float32),
        pltpu.SemaphoreType.DMA,
    ],
    compiler_params=_sc_params,
)
def _gs_kernel(xw_hbm, src_hbm, dst_hbm, out_hbm,
               src_v, dst_v, rows_v, zb_v, shared, sem):
    c = lax.axis_index("c")
    s = lax.axis_index("s")

    zeros16 = jnp.zeros((L,), jnp.float32)
    cols = D // L

    def zero_zb(t, carry):
        zb_v[t // cols, pl.ds((t % cols) * L, L)] = zeros16
        return carry

    lax.fori_loop(0, ZR * cols, zero_zb, 0)

    g0 = c * GPC + (s * GPC) // NS
    g1 = c * GPC + ((s + 1) * GPC) // NS

    for h in (0, 1):
        lo = h * HALF

        for k in range(AGG // NS // ZR):
            pltpu.sync_copy(zb_v, shared.at[pl.ds(s * (AGG // NS) + k * ZR, ZR)])
        plsc.subcore_barrier()

        def group_body(g, carry):
            rowb = g * GR
            pltpu.sync_copy(src_hbm.at[pl.ds(rowb, GR)], src_v)
            pltpu.sync_copy(dst_hbm.at[pl.ds(rowb, GR)], dst_v)
            for r in range(GR):
                for q in range(GB // L):
                    v = dst_v[r, pl.ds(q * L, L)]
                    m = (v >= lo) & (v < lo + HALF)
                    dst_v[r, pl.ds(q * L, L)] = jnp.where(m, v - lo, TRASH)
            for r in range(GR):
                pltpu.async_copy(xw_hbm.at[src_v.at[r]], rows_v, sem).wait()
                pltpu.sync_copy(rows_v, shared.at[dst_v.at[r]], add=True)
            return carry

        lax.fori_loop(g0, g1, group_body, 0)
        plsc.subcore_barrier()

        for k in range(NS):
            @pl.when(s == k)
            def _(k=k, h=h):
                pltpu.sync_copy(shared.at[pl.ds(k * OPT, OPT)],
                                out_hbm.at[h, c, pl.ds(k * OPT, OPT), :])
        plsc.subcore_barrier()


def _ep_body(p_ref, deg_ref, b_ref, out_ref):
    scale = lax.rsqrt(jnp.maximum(deg_ref[...], 1.0))
    p = p_ref[...]
    agg = (p[0, 0] + p[0, 1]) * scale
    out_ref[...] = jnp.maximum(agg + b_ref[...], 0.0)


def _epilogue(p, deg_in2d, b2d):
    return pl.pallas_call(
        _ep_body,
        grid=(N // RMM,),
        in_specs=[
            pl.BlockSpec((1, NC, RMM, D), lambda i: (i // 5, 0, i % 5, 0)),
            pl.BlockSpec((RMM, 1), lambda i: (i, 0)),
            pl.BlockSpec((1, D), lambda i: (0, 0)),
        ],
        out_specs=pl.BlockSpec((RMM, D), lambda i: (i, 0)),
        out_shape=jax.ShapeDtypeStruct((N, D), jnp.float32),
    )(p, deg_in2d, b2d)


def kernel(features, edge_index, W, b):
    srcf = edge_index[0]
    dstf = edge_index[1]
    dsrc, ddst = _deg_kernel(srcf, dstf)
    deg_out2d = dsrc[:N, None]
    deg_in2d = ddst[:N, None]
    xw = _mm(features, W, deg_out2d)
    src80 = srcf.reshape(E // GB, GB)
    dst80 = dstf.reshape(E // GB, GB)
    p = _gs_kernel(xw, src80, dst80)
    return _epilogue(p, deg_in2d, b[None, :])


# speedup vs baseline: 18.8686x; 1.4050x over previous
import functools

import jax
import jax.numpy as jnp
from jax import lax
from jax.experimental import pallas as pl
from jax.experimental.pallas import tpu as pltpu
from jax.experimental.pallas import tpu_sc as plsc

N = 10000
E = 320000
D = 128

NC = 2
NS = 16
L = 16

_mesh = plsc.VectorSubcoreMesh(core_axis_name="c", subcore_axis_name="s")
_sc_params = pltpu.CompilerParams(needs_layout_passes=False)

HSZ = 16384
EPT_DEG = E // NS
HPT = HSZ // NS


@functools.partial(
    pl.kernel,
    out_type=[
        jax.ShapeDtypeStruct((HSZ,), jnp.float32),
        jax.ShapeDtypeStruct((HSZ,), jnp.float32),
    ],
    mesh=_mesh,
    scratch_types=[
        pltpu.VMEM((EPT_DEG,), jnp.int32),
        pltpu.VMEM((HSZ,), jnp.float32),
        pltpu.VMEM((HPT,), jnp.float32),
        pltpu.VMEM((HPT,), jnp.float32),
        pltpu.VMEM_SHARED((NS * HSZ,), jnp.float32),
    ],
    compiler_params=_sc_params,
)
def _deg_kernel(src_hbm, dst_hbm, dsrc_hbm, ddst_hbm,
                idx_v, hist_v, acc_v, tmp_v, shared):
    c = lax.axis_index("c")
    s = lax.axis_index("s")

    zeros16 = jnp.zeros((L,), jnp.float32)
    ones16 = jnp.ones((L,), jnp.float32)

    def zero_hist(i, carry):
        hist_v[pl.ds(i * L, L)] = zeros16
        return carry

    lax.fori_loop(0, HSZ // L, zero_hist, 0)

    @pl.when(c == 0)
    def _():
        pltpu.sync_copy(src_hbm.at[pl.ds(s * EPT_DEG, EPT_DEG)], idx_v)

    @pl.when(c == 1)
    def _():
        pltpu.sync_copy(dst_hbm.at[pl.ds(s * EPT_DEG, EPT_DEG)], idx_v)

    def accum(i, carry):
        idx = idx_v[pl.ds(i * L, L)]
        plsc.addupdate_scatter(hist_v, [idx], ones16)
        return carry

    lax.fori_loop(0, EPT_DEG // L, accum, 0)

    pltpu.sync_copy(hist_v, shared.at[pl.ds(s * HSZ, HSZ)])
    plsc.subcore_barrier()

    def zero_acc(i, carry):
        acc_v[pl.ds(i * L, L)] = zeros16
        return carry

    lax.fori_loop(0, HPT // L, zero_acc, 0)

    def combine(k, carry):
        pltpu.sync_copy(shared.at[pl.ds(k * HSZ + s * HPT, HPT)], tmp_v)

        def add_vec(i, carry2):
            j = i * L
            acc_v[pl.ds(j, L)] = acc_v[pl.ds(j, L)] + tmp_v[pl.ds(j, L)]
            return carry2

        lax.fori_loop(0, HPT // L, add_vec, 0)
        return carry

    lax.fori_loop(0, NS, combine, 0)

    @pl.when(c == 0)
    def _():
        pltpu.sync_copy(acc_v, dsrc_hbm.at[pl.ds(s * HPT, HPT)])

    @pl.when(c == 1)
    def _():
        pltpu.sync_copy(acc_v, ddst_hbm.at[pl.ds(s * HPT, HPT)])


RMM = 1000


def _mm_body(f_ref, w_ref, deg_ref, xw_ref):
    scale = lax.rsqrt(jnp.maximum(deg_ref[...], 1.0))
    xw_ref[...] = jnp.dot(f_ref[...], w_ref[...],
                          preferred_element_type=jnp.float32) * scale


def _mm(features, W, deg_out2d):
    return pl.pallas_call(
        _mm_body,
        grid=(N // RMM,),
        in_specs=[
            pl.BlockSpec((RMM, D), lambda i: (i, 0)),
            pl.BlockSpec((D, D), lambda i: (0, 0)),
            pl.BlockSpec((RMM, 1), lambda i: (i, 0)),
        ],
        out_specs=pl.BlockSpec((RMM, D), lambda i: (i, 0)),
        out_shape=jax.ShapeDtypeStruct((N, D), jnp.float32),
    )(features, W, deg_out2d)


GB = 80
GR = 8
GE = GB * GR
NG = E // GE
GPC = NG // NC
HALF = 5000
AGG = 6144
TRASH = 5632
ZR = 96
OPT = 5120 // NS


NBUF = 4


@functools.partial(
    pl.kernel,
    out_type=jax.ShapeDtypeStruct((2, NC, 5120, D), jnp.float32),
    mesh=_mesh,
    scratch_types=[
        pltpu.VMEM((2, GR, GB), jnp.int32),
        pltpu.VMEM((2, GR, GB), jnp.int32),
        pltpu.VMEM((GB, D), jnp.float32),
        pltpu.VMEM((GB, D), jnp.float32),
        pltpu.VMEM((GB, D), jnp.float32),
        pltpu.VMEM((GB, D), jnp.float32),
        pltpu.VMEM((ZR, D), jnp.float32),
        pltpu.VMEM_SHARED((AGG, D), jnp.float32),
        pltpu.SemaphoreType.DMA,
        pltpu.SemaphoreType.DMA,
        pltpu.SemaphoreType.DMA,
        pltpu.SemaphoreType.DMA,
        pltpu.SemaphoreType.DMA,
    ],
    compiler_params=_sc_params,
)
def _gs_kernel(xw_hbm, src_hbm, dst_hbm, out_hbm,
               src_v, dst_v, r0_v, r1_v, r2_v, r3_v, zb_v, shared,
               sg0, sg1, sg2, sg3, si):
    c = lax.axis_index("c")
    s = lax.axis_index("s")
    rows = (r0_v, r1_v, r2_v, r3_v)
    sgs = (sg0, sg1, sg2, sg3)

    zeros16 = jnp.zeros((L,), jnp.float32)
    cols = D // L

    def zero_zb(t, carry):
        zb_v[t // cols, pl.ds((t % cols) * L, L)] = zeros16
        return carry

    lax.fori_loop(0, ZR * cols, zero_zb, 0)

    g0 = c * GPC + (s * GPC) // NS
    g1 = c * GPC + ((s + 1) * GPC) // NS

    def issue_idx(g):
        par = (g - g0) % 2
        pltpu.async_copy(src_hbm.at[pl.ds(g * GR, GR)], src_v.at[par], si)
        pltpu.async_copy(dst_hbm.at[pl.ds(g * GR, GR)], dst_v.at[par], si)

    for h in (0, 1):
        lo = h * HALF

        for k in range(AGG // NS // ZR):
            pltpu.sync_copy(zb_v, shared.at[pl.ds(s * (AGG // NS) + k * ZR, ZR)])
        plsc.subcore_barrier()

        issue_idx(g0)

        def group_body(g, carry):
            par = (g - g0) % 2
            pltpu.make_async_copy(src_hbm.at[pl.ds(g * GR, GR)],
                                  src_v.at[par], si).wait()
            pltpu.make_async_copy(dst_hbm.at[pl.ds(g * GR, GR)],
                                  dst_v.at[par], si).wait()

            @pl.when(g + 1 < g1)
            def _():
                issue_idx(g + 1)

            for r in range(GR):
                for q in range(GB // L):
                    v = dst_v[par, r, pl.ds(q * L, L)]
                    m = (v >= lo) & (v < lo + HALF)
                    dst_v[par, r, pl.ds(q * L, L)] = jnp.where(m, v - lo, TRASH)

            for r in range(NBUF):
                pltpu.async_copy(xw_hbm.at[src_v.at[par, r]], rows[r], sgs[r])
            for r in range(GR):
                slot = r % NBUF
                pltpu.make_async_copy(xw_hbm.at[src_v.at[par, r]],
                                      rows[slot], sgs[slot]).wait()
                pltpu.sync_copy(rows[slot], shared.at[dst_v.at[par, r]],
                                add=True)
                if r + NBUF < GR:
                    pltpu.async_copy(xw_hbm.at[src_v.at[par, r + NBUF]],
                                     rows[slot], sgs[slot])
            return carry

        lax.fori_loop(g0, g1, group_body, 0)
        plsc.subcore_barrier()

        for k in range(NS):
            @pl.when(s == k)
            def _(k=k, h=h):
                pltpu.sync_copy(shared.at[pl.ds(k * OPT, OPT)],
                                out_hbm.at[h, c, pl.ds(k * OPT, OPT), :])
        plsc.subcore_barrier()


def _ep_body(p_ref, deg_ref, b_ref, out_ref):
    scale = lax.rsqrt(jnp.maximum(deg_ref[...], 1.0))
    p = p_ref[...]
    agg = (p[0, 0] + p[0, 1]) * scale
    out_ref[...] = jnp.maximum(agg + b_ref[...], 0.0)


def _epilogue(p, deg_in2d, b2d):
    return pl.pallas_call(
        _ep_body,
        grid=(N // RMM,),
        in_specs=[
            pl.BlockSpec((1, NC, RMM, D), lambda i: (i // 5, 0, i % 5, 0)),
            pl.BlockSpec((RMM, 1), lambda i: (i, 0)),
            pl.BlockSpec((1, D), lambda i: (0, 0)),
        ],
        out_specs=pl.BlockSpec((RMM, D), lambda i: (i, 0)),
        out_shape=jax.ShapeDtypeStruct((N, D), jnp.float32),
    )(p, deg_in2d, b2d)


def kernel(features, edge_index, W, b):
    srcf = edge_index[0]
    dstf = edge_index[1]
    dsrc, ddst = _deg_kernel(srcf, dstf)
    deg_out2d = dsrc[:N, None]
    deg_in2d = ddst[:N, None]
    xw = _mm(features, W, deg_out2d)
    src80 = srcf.reshape(E // GB, GB)
    dst80 = dstf.reshape(E // GB, GB)
    p = _gs_kernel(xw, src80, dst80)
    return _epilogue(p, deg_in2d, b[None, :])
